# baseline (speedup 1.0000x reference)
import functools

import jax
import jax.numpy as jnp
from jax import lax
from jax.experimental import pallas as pl
from jax.experimental.pallas import tpu as pltpu
from jax.experimental.pallas import tpu_sc as plsc

_K = 8192
_D = 256
_N = 8192
_T = 1024
_KC = 1024
_GR = 8
_COMMIT = 0.25


def _vq_body(z_ref, w_ref, idx_ref, loss_ref):
    zt = z_ref[0]
    zn = jnp.sum(zt * zt, axis=0, keepdims=True)
    zm2 = -2.0 * zt

    subf = jax.lax.broadcasted_iota(jnp.int32, (_GR, _T), 0).astype(
        jnp.float32)

    M = jnp.full((_GR, _T), jnp.inf, jnp.float32)
    A = jnp.zeros((_GR, _T), jnp.float32)
    for c in range(_K // _KC):
        wc = w_ref[c * _KC:(c + 1) * _KC, :]
        wnc = jnp.sum(wc * wc, axis=1, keepdims=True)
        m2 = jax.lax.dot_general(wc, zm2, (((1,), (0,)), ((), ())),
                                 preferred_element_type=jnp.float32)
        for gg in range(_KC // _GR):
            sg = ((zn + m2[gg * _GR:(gg + 1) * _GR, :])
                  + wnc[gg * _GR:(gg + 1) * _GR, :])
            colf = subf + jnp.float32(c * _KC + gg * _GR)
            upd = sg < M
            M = jnp.where(upd, sg, M)
            A = jnp.where(upd, colf, A)

    gmin = jnp.min(M, axis=0, keepdims=True)
    key = (M - gmin) * jnp.float32(2.0 ** 60) + A
    idx_ref[0] = jnp.min(key, axis=0, keepdims=True).astype(jnp.int32)

    part = jnp.sum(gmin).reshape(1, 1)

    @pl.when(pl.program_id(0) == 0)
    def _():
        loss_ref[...] = part

    @pl.when(pl.program_id(0) != 0)
    def _():
        loss_ref[...] = loss_ref[...] + part


_SC_INFO = plsc.get_sparse_core_info()
_NW = _SC_INFO.num_cores * _SC_INFO.num_subcores
_BPW = _N // _NW


@functools.partial(
    pl.kernel,
    mesh=plsc.VectorSubcoreMesh(core_axis_name="c", subcore_axis_name="s"),
    out_type=jax.ShapeDtypeStruct((_N, _D), jnp.float32),
    scratch_types=[
        pltpu.VMEM((_BPW,), jnp.int32),
        pltpu.VMEM((_BPW, _D), jnp.float32),
        pltpu.SemaphoreType.DMA,
    ],
)
def _sc_gather(table_hbm, idx_hbm, out_hbm, idx_v, rows_v, sem):
    wid = lax.axis_index("s") * _SC_INFO.num_cores + lax.axis_index("c")
    base = wid * _BPW
    pltpu.sync_copy(idx_hbm.at[pl.ds(base, _BPW)], idx_v)
    pltpu.async_copy(table_hbm.at[idx_v], rows_v, sem).wait()
    pltpu.sync_copy(rows_v, out_hbm.at[pl.ds(base, _BPW)])


def kernel(z_e, weight):
    B, D, H, W = z_e.shape
    z3 = z_e.reshape(B, D, H * W)

    idx, loss_part = pl.pallas_call(
        _vq_body,
        grid=(B,),
        in_specs=[
            pl.BlockSpec((1, _D, _T), lambda i: (i, 0, 0)),
            pl.BlockSpec((_K, _D), lambda i: (0, 0)),
        ],
        out_specs=[
            pl.BlockSpec((1, 1, _T), lambda i: (i, 0, 0)),
            pl.BlockSpec((1, 1), lambda i: (0, 0)),
        ],
        out_shape=[
            jax.ShapeDtypeStruct((B, 1, _T), jnp.int32),
            jax.ShapeDtypeStruct((1, 1), jnp.float32),
        ],
    )(z3, weight)

    zq_flat = _sc_gather(weight, idx.reshape(-1))

    commit_loss = loss_part[0, 0] / (_N * _D)
    zq_sp = jnp.transpose(zq_flat.reshape(B, H, W, D), (0, 3, 1, 2))
    z_q = z_e + jax.lax.stop_gradient(zq_sp - z_e)
    return (z_q, _COMMIT * commit_loss, idx.reshape(B, H, W))


# speedup vs baseline: 1.0241x; 1.0241x over previous
import functools

import jax
import jax.numpy as jnp
from jax import lax
from jax.experimental import pallas as pl
from jax.experimental.pallas import tpu as pltpu
from jax.experimental.pallas import tpu_sc as plsc

_K = 8192
_D = 256
_N = 8192
_T = 1024
_KC = 4096
_GR = 8
_COMMIT = 0.25


def _vq_body(z_ref, w_ref, idx_ref, loss_ref):
    zt = z_ref[0]
    zn = jnp.sum(zt * zt, axis=0, keepdims=True)
    zm2 = -2.0 * zt

    subf = jax.lax.broadcasted_iota(jnp.int32, (_GR, _T), 0).astype(
        jnp.float32)

    M = jnp.full((_GR, _T), jnp.inf, jnp.float32)
    A = jnp.zeros((_GR, _T), jnp.float32)
    for c in range(_K // _KC):
        wc = w_ref[c * _KC:(c + 1) * _KC, :]
        wnc = jnp.sum(wc * wc, axis=1, keepdims=True)
        m2 = jax.lax.dot_general(wc, zm2, (((1,), (0,)), ((), ())),
                                 preferred_element_type=jnp.float32)
        for gg in range(_KC // _GR):
            sg = ((zn + m2[gg * _GR:(gg + 1) * _GR, :])
                  + wnc[gg * _GR:(gg + 1) * _GR, :])
            colf = subf + jnp.float32(c * _KC + gg * _GR)
            upd = sg < M
            M = jnp.where(upd, sg, M)
            A = jnp.where(upd, colf, A)

    gmin = jnp.min(M, axis=0, keepdims=True)
    key = (M - gmin) * jnp.float32(2.0 ** 60) + A
    idx_ref[0] = jnp.min(key, axis=0, keepdims=True).astype(jnp.int32)

    part = jnp.sum(gmin).reshape(1, 1)

    @pl.when(pl.program_id(0) == 0)
    def _():
        loss_ref[...] = part

    @pl.when(pl.program_id(0) != 0)
    def _():
        loss_ref[...] = loss_ref[...] + part


_SC_INFO = plsc.get_sparse_core_info()
_NW = _SC_INFO.num_cores * _SC_INFO.num_subcores
_BPW = _N // _NW


@functools.partial(
    pl.kernel,
    mesh=plsc.VectorSubcoreMesh(core_axis_name="c", subcore_axis_name="s"),
    out_type=jax.ShapeDtypeStruct((_N, _D), jnp.float32),
    scratch_types=[
        pltpu.VMEM((_BPW,), jnp.int32),
        pltpu.VMEM((_BPW, _D), jnp.float32),
        pltpu.SemaphoreType.DMA,
    ],
)
def _sc_gather(table_hbm, idx_hbm, out_hbm, idx_v, rows_v, sem):
    wid = lax.axis_index("s") * _SC_INFO.num_cores + lax.axis_index("c")
    base = wid * _BPW
    pltpu.sync_copy(idx_hbm.at[pl.ds(base, _BPW)], idx_v)
    pltpu.async_copy(table_hbm.at[idx_v], rows_v, sem).wait()
    pltpu.sync_copy(rows_v, out_hbm.at[pl.ds(base, _BPW)])


def kernel(z_e, weight):
    B, D, H, W = z_e.shape
    z3 = z_e.reshape(B, D, H * W)

    idx, loss_part = pl.pallas_call(
        _vq_body,
        grid=(B,),
        in_specs=[
            pl.BlockSpec((1, _D, _T), lambda i: (i, 0, 0)),
            pl.BlockSpec((_K, _D), lambda i: (0, 0)),
        ],
        out_specs=[
            pl.BlockSpec((1, 1, _T), lambda i: (i, 0, 0)),
            pl.BlockSpec((1, 1), lambda i: (0, 0)),
        ],
        out_shape=[
            jax.ShapeDtypeStruct((B, 1, _T), jnp.int32),
            jax.ShapeDtypeStruct((1, 1), jnp.float32),
        ],
    )(z3, weight)

    zq_flat = _sc_gather(weight, idx.reshape(-1))

    commit_loss = loss_part[0, 0] / (_N * _D)
    zq_sp = jnp.transpose(zq_flat.reshape(B, H, W, D), (0, 3, 1, 2))
    z_q = z_e + jax.lax.stop_gradient(zq_sp - z_e)
    return (z_q, _COMMIT * commit_loss, idx.reshape(B, H, W))


# speedup vs baseline: 1.0350x; 1.0106x over previous
import functools

import jax
import jax.numpy as jnp
from jax import lax
from jax.experimental import pallas as pl
from jax.experimental.pallas import tpu as pltpu
from jax.experimental.pallas import tpu_sc as plsc

_K = 8192
_D = 256
_N = 8192
_T = 1024
_KC = 8192
_GR = 8
_COMMIT = 0.25


def _vq_body(z_ref, w_ref, idx_ref, loss_ref):
    zt = z_ref[0]
    zn = jnp.sum(zt * zt, axis=0, keepdims=True)
    zm2 = -2.0 * zt

    subf = jax.lax.broadcasted_iota(jnp.int32, (_GR, _T), 0).astype(
        jnp.float32)

    M = jnp.full((_GR, _T), jnp.inf, jnp.float32)
    A = jnp.zeros((_GR, _T), jnp.float32)
    for c in range(_K // _KC):
        wc = w_ref[c * _KC:(c + 1) * _KC, :]
        wnc = jnp.sum(wc * wc, axis=1, keepdims=True)
        m2 = jax.lax.dot_general(wc, zm2, (((1,), (0,)), ((), ())),
                                 preferred_element_type=jnp.float32)
        for gg in range(_KC // _GR):
            sg = ((zn + m2[gg * _GR:(gg + 1) * _GR, :])
                  + wnc[gg * _GR:(gg + 1) * _GR, :])
            colf = subf + jnp.float32(c * _KC + gg * _GR)
            upd = sg < M
            M = jnp.where(upd, sg, M)
            A = jnp.where(upd, colf, A)

    gmin = jnp.min(M, axis=0, keepdims=True)
    key = (M - gmin) * jnp.float32(2.0 ** 60) + A
    idx_ref[0] = jnp.min(key, axis=0, keepdims=True).astype(jnp.int32)

    part = jnp.sum(gmin).reshape(1, 1)

    @pl.when(pl.program_id(0) == 0)
    def _():
        loss_ref[...] = part

    @pl.when(pl.program_id(0) != 0)
    def _():
        loss_ref[...] = loss_ref[...] + part


_SC_INFO = plsc.get_sparse_core_info()
_NW = _SC_INFO.num_cores * _SC_INFO.num_subcores
_BPW = _N // _NW


@functools.partial(
    pl.kernel,
    mesh=plsc.VectorSubcoreMesh(core_axis_name="c", subcore_axis_name="s"),
    out_type=jax.ShapeDtypeStruct((_N, _D), jnp.float32),
    scratch_types=[
        pltpu.VMEM((_BPW,), jnp.int32),
        pltpu.VMEM((_BPW, _D), jnp.float32),
        pltpu.SemaphoreType.DMA,
    ],
)
def _sc_gather(table_hbm, idx_hbm, out_hbm, idx_v, rows_v, sem):
    wid = lax.axis_index("s") * _SC_INFO.num_cores + lax.axis_index("c")
    base = wid * _BPW
    pltpu.sync_copy(idx_hbm.at[pl.ds(base, _BPW)], idx_v)
    pltpu.async_copy(table_hbm.at[idx_v], rows_v, sem).wait()
    pltpu.sync_copy(rows_v, out_hbm.at[pl.ds(base, _BPW)])


def kernel(z_e, weight):
    B, D, H, W = z_e.shape
    z3 = z_e.reshape(B, D, H * W)

    idx, loss_part = pl.pallas_call(
        _vq_body,
        grid=(B,),
        in_specs=[
            pl.BlockSpec((1, _D, _T), lambda i: (i, 0, 0)),
            pl.BlockSpec((_K, _D), lambda i: (0, 0)),
        ],
        out_specs=[
            pl.BlockSpec((1, 1, _T), lambda i: (i, 0, 0)),
            pl.BlockSpec((1, 1), lambda i: (0, 0)),
        ],
        out_shape=[
            jax.ShapeDtypeStruct((B, 1, _T), jnp.int32),
            jax.ShapeDtypeStruct((1, 1), jnp.float32),
        ],
    )(z3, weight)

    zq_flat = _sc_gather(weight, idx.reshape(-1))

    commit_loss = loss_part[0, 0] / (_N * _D)
    zq_sp = jnp.transpose(zq_flat.reshape(B, H, W, D), (0, 3, 1, 2))
    z_q = z_e + jax.lax.stop_gradient(zq_sp - z_e)
    return (z_q, _COMMIT * commit_loss, idx.reshape(B, H, W))
